# baseline (speedup 1.0000x reference)
import jax
import jax.numpy as jnp
from jax import lax
from jax.experimental import pallas as pl
from jax.experimental.pallas import tpu as pltpu

_N = 4096
_H = 2048
_BLK = 256
_G = _N // _BLK
_ETA = 1000000.0
_NT = (((1,), (1,)), ((), ()))


def _body(Q_ref, AT_ref, x_ref, y_ref, c_ref, b_ref, il_ref, iu_ref,
          l_ref, u_ref, o_ref, acc_ref, qx_ref, aty_ref):
    i = pl.program_id(0)
    j = pl.program_id(1)

    @pl.when((i == 0) & (j == 0))
    def _init():
        acc_ref[...] = jnp.zeros((1, _BLK), jnp.float32)

    xh = x_ref[:, pl.ds(j * _H, _H)]
    yh = y_ref[:, pl.ds(j * _H, _H)]
    qp = lax.dot_general(xh, Q_ref[...], _NT,
                         preferred_element_type=jnp.float32)
    ap = lax.dot_general(yh, AT_ref[...], _NT,
                         preferred_element_type=jnp.float32)

    @pl.when(j == 0)
    def _first_half():
        qx_ref[...] = qp
        aty_ref[...] = ap

    @pl.when(j == 1)
    def _second_half():
        qx = qx_ref[...] + qp
        aty = aty_ref[...] + ap
        sl = pl.ds(i * _BLK, _BLK)
        xb = x_ref[:, sl]
        cb = c_ref[:, sl]
        pg = cb - aty + qx
        rc = (jnp.maximum(pg, 0.0) * il_ref[:, sl]
              - jnp.maximum(-pg, 0.0) * iu_ref[:, sl])
        rcv = jnp.where(rc > 0.0, l_ref[:, sl], u_ref[:, sl]) * rc
        contrib = xb * qx + cb * xb - b_ref[:, sl] * y_ref[:, sl] - rcv
        acc_ref[...] = acc_ref[...] + contrib

    @pl.when((i == _G - 1) & (j == 1))
    def _fin():
        o_ref[...] = jnp.full((1, 1), jnp.abs(jnp.sum(acc_ref[...])) / _ETA,
                              dtype=jnp.float32)


def kernel(Q, A, AT, b, c, x, y, Iy, il, iu, l, u):
    del A, Iy
    xT = x.reshape(1, _N)
    yT = y.reshape(1, _N)
    cT = c.reshape(1, _N)
    bT = b.reshape(1, _N)
    ilT = il.reshape(1, _N)
    iuT = iu.reshape(1, _N)
    lT = l.reshape(1, _N)
    uT = u.reshape(1, _N)
    row = pl.BlockSpec((1, _N), lambda i, j: (0, 0))
    half = pl.BlockSpec((_BLK, _H), lambda i, j: (i, j))
    out = pl.pallas_call(
        _body,
        grid=(_G, 2),
        in_specs=[half, half,
                  row, row, row, row, row, row, row, row],
        out_specs=pl.BlockSpec((1, 1), lambda i, j: (0, 0)),
        out_shape=jax.ShapeDtypeStruct((1, 1), jnp.float32),
        scratch_shapes=[pltpu.VMEM((1, _BLK), jnp.float32),
                        pltpu.VMEM((1, _BLK), jnp.float32),
                        pltpu.VMEM((1, _BLK), jnp.float32)],
        compiler_params=pltpu.CompilerParams(
            dimension_semantics=("arbitrary", "arbitrary")),
    )(Q, AT, xT, yT, cT, bT, ilT, iuT, lT, uT)
    return out
